# baseline (speedup 1.0000x reference)
import jax
import jax.numpy as jnp
from jax.experimental import pallas as pl
from jax.experimental.pallas import tpu as pltpu


def _edge_inject(x, dst, edge_attr, p):
    h = jax.nn.relu(edge_attr @ p["W1"] + p["b1"])
    msg = h @ p["W2"] + p["b2"]
    msg = msg * jax.nn.sigmoid(edge_attr[:, 2:3])
    out = jax.ops.segment_sum(msg, dst, num_segments=x.shape[0])
    return x + out


def _sage(x, src, dst, p, inv_cnt):
    agg = jax.ops.segment_sum(x[src], dst, num_segments=x.shape[0])
    mean = agg * inv_cnt[:, None]
    return mean @ p["Wl"] + p["bl"] + x @ p["Wr"]


def _bn(x, p, eps=1e-5):
    m = jnp.mean(x, axis=0)
    v = jnp.var(x, axis=0)
    return p["g"] * (x - m) / jnp.sqrt(v + eps) + p["b"]


def _head_kernel(h_ref, ws_ref, bs_ref, wf1_ref, bf1_ref, wf2_ref, bf2_ref,
                 wr1_ref, br1_ref, wr2_ref, br2_ref, fire_ref, rate_ref):
    h = h_ref[...]
    s = jnp.maximum(h @ ws_ref[...] + bs_ref[...], 0.0)
    f = jnp.maximum(s @ wf1_ref[...] + bf1_ref[...], 0.0)
    fire = f @ wf2_ref[...] + bf2_ref[...]
    r = jnp.maximum(s @ wr1_ref[...] + br1_ref[...], 0.0)
    rate = jnp.maximum(r @ wr2_ref[...] + br2_ref[...], 0.0)
    fire_ref[...] = fire
    rate_ref[...] = rate


def _head(h, params):
    n = h.shape[0]
    blk = 10000
    grid = (n // blk,)
    out_shape = (jax.ShapeDtypeStruct((n, 1), jnp.float32),
                 jax.ShapeDtypeStruct((n, 1), jnp.float32))
    wspec = pl.BlockSpec(lambda i: (0, 0))
    bspec = pl.BlockSpec(lambda i: (0,))
    fire, rate = pl.pallas_call(
        _head_kernel,
        grid=grid,
        in_specs=[pl.BlockSpec((blk, h.shape[1]), lambda i: (i, 0)),
                  pl.BlockSpec((32, 16), wspec.index_map),
                  pl.BlockSpec((16,), bspec.index_map),
                  pl.BlockSpec((16, 8), wspec.index_map),
                  pl.BlockSpec((8,), bspec.index_map),
                  pl.BlockSpec((8, 1), wspec.index_map),
                  pl.BlockSpec((1,), bspec.index_map),
                  pl.BlockSpec((16, 8), wspec.index_map),
                  pl.BlockSpec((8,), bspec.index_map),
                  pl.BlockSpec((8, 1), wspec.index_map),
                  pl.BlockSpec((1,), bspec.index_map)],
        out_specs=(pl.BlockSpec((blk, 1), lambda i: (i, 0)),
                   pl.BlockSpec((blk, 1), lambda i: (i, 0))),
        out_shape=out_shape,
    )(h, params["shared"]["W"], params["shared"]["b"],
      params["fire1"]["W"], params["fire1"]["b"],
      params["fire2"]["W"], params["fire2"]["b"],
      params["rate1"]["W"], params["rate1"]["b"],
      params["rate2"]["W"], params["rate2"]["b"])
    return fire[:, 0], rate[:, 0]


def kernel(x, edge_index, edge_attr, params):
    src, dst = edge_index[0], edge_index[1]
    n = x.shape[0]
    cnt = jax.ops.segment_sum(jnp.ones((src.shape[0],), x.dtype), dst,
                              num_segments=n)
    inv_cnt = 1.0 / jnp.maximum(cnt, 1.0)
    h = _edge_inject(x, dst, edge_attr, params["inj1"])
    h = jax.nn.relu(_bn(_sage(h, src, dst, params["conv1"], inv_cnt), params["bn1"]))
    h = _edge_inject(h, dst, edge_attr, params["inj2"])
    h = jax.nn.relu(_bn(_sage(h, src, dst, params["conv2"], inv_cnt), params["bn2"]))
    h = _edge_inject(h, dst, edge_attr, params["inj3"])
    h = jax.nn.relu(_bn(_sage(h, src, dst, params["conv3"], inv_cnt), params["bn3"]))
    return _head(h, params)


# speedup vs baseline: 1.6445x; 1.6445x over previous
import functools

import jax
import jax.numpy as jnp
from jax import lax
from jax.experimental import pallas as pl
from jax.experimental.pallas import tpu as pltpu
from jax.experimental.pallas import tpu_sc as plsc

_N = 100000
_E = 1600000
_EP = 1638400
_R = _EP // 128
_HALF = _R // 2
_TROWS = _HALF // 16
_BATCH = 8
_NB = _TROWS // _BATCH
_NACC = 100096
_TACC = _NACC // 16
_ZCH = _TACC // 2
_RW = _BATCH * 128


def _sc_body(S, gather, refs):
    i = 0
    pays = refs[i:i + S]; i += S
    if gather:
        src2 = refs[i]; i += 1
    dst2 = refs[i]; i += 1
    out = refs[i]; i += 1
    acc = refs[i]; i += 1
    dbuf = refs[i]; i += 1
    if gather:
        sbuf = refs[i]; i += 1
    rows = refs[i]; i += 1
    sem_g = refs[i]; i += 1
    sem_s = refs[i]; i += 1

    c = lax.axis_index("c")
    s = lax.axis_index("s")

    z16 = jnp.zeros((16,), jnp.float32)
    base = c * _HALF + s * _TROWS

    for f in range(S):
        pay = pays[f]

        def zb(k, carry):
            rows[k, :] = z16
            return carry

        lax.fori_loop(0, _RW, zb, 0)
        for k in range(6):
            pltpu.sync_copy(rows, acc.at[pl.ds(s * _TACC + k * _RW, _RW)])
        pltpu.sync_copy(rows.at[pl.ds(0, _TACC - 6 * _RW)],
                        acc.at[pl.ds(s * _TACC + 6 * _RW, _TACC - 6 * _RW)])
        plsc.subcore_barrier()

        def batch_body(b, carry):
            row0 = base + b * _BATCH
            pltpu.sync_copy(dst2.at[pl.ds(row0, _BATCH)], dbuf)
            if gather:
                pltpu.sync_copy(src2.at[pl.ds(row0, _BATCH)], sbuf)
                descs = [pltpu.async_copy(pay.at[sbuf.at[j]],
                                          rows.at[pl.ds(j * 128, 128)], sem_g)
                         for j in range(_BATCH)]
                for d in descs:
                    d.wait()
            else:
                pltpu.sync_copy(pay.at[pl.ds(row0 * 128, _BATCH * 128)], rows)
            descs = [pltpu.async_copy(rows.at[pl.ds(j * 128, 128)],
                                      acc.at[dbuf.at[j]], sem_s, add=True)
                     for j in range(_BATCH)]
            for d in descs:
                d.wait()
            return carry

        lax.fori_loop(0, _NB, batch_body, 0)
        plsc.subcore_barrier()

        off = (c * S + f) * _NACC + s * _TACC
        for k in range(2):
            pltpu.sync_copy(acc.at[pl.ds(s * _TACC + k * _ZCH, _ZCH)],
                            out.at[pl.ds(off + k * _ZCH, _ZCH)])
        plsc.subcore_barrier()


_SC_CACHE = {}


def _sc_segsum(S, gather):
    key = (S, gather)
    if key not in _SC_CACHE:
        scratch = [pltpu.VMEM_SHARED((_NACC, 16), jnp.float32),
                   pltpu.VMEM((_BATCH, 128), jnp.int32)]
        if gather:
            scratch.append(pltpu.VMEM((_BATCH, 128), jnp.int32))
        scratch += [pltpu.VMEM((_BATCH * 128, 16), jnp.float32),
                    pltpu.SemaphoreType.DMA,
                    pltpu.SemaphoreType.DMA]
        fn = pl.kernel(
            lambda *refs: _sc_body(S, gather, refs),
            out_type=jax.ShapeDtypeStruct((2 * S * _NACC, 16), jnp.float32),
            mesh=plsc.VectorSubcoreMesh(core_axis_name="c",
                                        subcore_axis_name="s"),
            scratch_types=scratch,
            compiler_params=pltpu.CompilerParams(use_tc_tiling_on_sc=False),
            name=f"segsum_s{S}_{'g' if gather else 'l'}",
        )
        _SC_CACHE[key] = fn
    return _SC_CACHE[key]


def _lin_segsum(pays, dst2):
    S = len(pays)
    padz = jnp.zeros((_EP - _E, 16), jnp.float32)
    pads = [jnp.concatenate([p, padz], axis=0) for p in pays]
    part = _sc_segsum(S, False)(*pads, dst2)
    return part.reshape(2, S, _NACC, 16)[:, :, :_N].sum(axis=0)


def _gat_segsum(tables, src2, dst2):
    S = len(tables)
    part = _sc_segsum(S, True)(*tables, src2, dst2)
    return part.reshape(2, S, _NACC, 16)[:, :, :_N].sum(axis=0)


def _slices(m, S):
    return [m[:, 16 * f:16 * f + 16] for f in range(S)]


def _unslice(t):
    return t.transpose(1, 0, 2).reshape(t.shape[1], -1)


def _bn(z, p, eps=1e-5):
    m = jnp.mean(z, axis=0)
    v = jnp.var(z, axis=0)
    return p["g"] * (z - m) / jnp.sqrt(v + eps) + p["b"]


def _head_kernel(h_ref, ws_ref, bs_ref, wf1_ref, bf1_ref, wf2_ref, bf2_ref,
                 wr1_ref, br1_ref, wr2_ref, br2_ref, fire_ref, rate_ref):
    h = h_ref[...]
    sh = jnp.maximum(h @ ws_ref[...] + bs_ref[...], 0.0)
    f = jnp.maximum(sh @ wf1_ref[...] + bf1_ref[...], 0.0)
    fire = f @ wf2_ref[...] + bf2_ref[...]
    r = jnp.maximum(sh @ wr1_ref[...] + br1_ref[...], 0.0)
    rate = jnp.maximum(r @ wr2_ref[...] + br2_ref[...], 0.0)
    fire_ref[...] = fire
    rate_ref[...] = rate


def _head(h, params):
    n = h.shape[0]
    blk = 10000
    grid = (n // blk,)
    out_shape = (jax.ShapeDtypeStruct((n, 1), jnp.float32),
                 jax.ShapeDtypeStruct((n, 1), jnp.float32))
    wmap = lambda i: (0, 0)
    bmap = lambda i: (0,)
    fire, rate = pl.pallas_call(
        _head_kernel,
        grid=grid,
        in_specs=[pl.BlockSpec((blk, h.shape[1]), lambda i: (i, 0)),
                  pl.BlockSpec((32, 16), wmap),
                  pl.BlockSpec((16,), bmap),
                  pl.BlockSpec((16, 8), wmap),
                  pl.BlockSpec((8,), bmap),
                  pl.BlockSpec((8, 1), wmap),
                  pl.BlockSpec((1,), bmap),
                  pl.BlockSpec((16, 8), wmap),
                  pl.BlockSpec((8,), bmap),
                  pl.BlockSpec((8, 1), wmap),
                  pl.BlockSpec((1,), bmap)],
        out_specs=(pl.BlockSpec((blk, 1), lambda i: (i, 0)),
                   pl.BlockSpec((blk, 1), lambda i: (i, 0))),
        out_shape=out_shape,
    )(h, params["shared"]["W"], params["shared"]["b"],
      params["fire1"]["W"], params["fire1"]["b"],
      params["fire2"]["W"], params["fire2"]["b"],
      params["rate1"]["W"], params["rate1"]["b"],
      params["rate2"]["W"], params["rate2"]["b"])
    return fire[:, 0], rate[:, 0]


def kernel(x, edge_index, edge_attr, params):
    src, dst = edge_index[0], edge_index[1]
    npad = _EP - _E
    dst_pad = _N + jnp.arange(npad, dtype=jnp.int32) % (_NACC - _N)
    src_pad = jnp.arange(npad, dtype=jnp.int32) % _N
    src2 = jnp.concatenate([src, src_pad]).reshape(_R, 128)
    dst2 = jnp.concatenate([dst, dst_pad]).reshape(_R, 128)
    g = jax.nn.sigmoid(edge_attr[:, 2:3])

    p1 = params["inj1"]
    h1e = jax.nn.relu(edge_attr @ p1["W1"] + p1["b1"]) * g
    pay1 = jnp.concatenate(
        [h1e, jnp.ones((_E, 1), jnp.float32), g,
         jnp.zeros((_E, 9), jnp.float32)], axis=1)
    acc1 = _lin_segsum([pay1], dst2)[0]
    s_hg1 = acc1[:, :5]
    cnt = acc1[:, 5]
    sg = acc1[:, 6:7]
    inv = (1.0 / jnp.maximum(cnt, 1.0))[:, None]

    h1 = x + s_hg1 @ p1["W2"] + sg * p1["b2"]
    c1 = params["conv1"]
    tab1 = jnp.pad(h1, ((0, 0), (0, 11)))
    agg1 = _gat_segsum([tab1], src2, dst2)[0][:, :5]
    z = (agg1 * inv) @ c1["Wl"] + c1["bl"] + h1 @ c1["Wr"]
    h = jax.nn.relu(_bn(z, params["bn1"]))

    for inj_key, conv_key, bn_key in (("inj2", "conv2", "bn2"),):
        pi = params[inj_key]
        he = jax.nn.relu(edge_attr @ pi["W1"] + pi["b1"]) * g
        s_hg = _unslice(_lin_segsum(_slices(he, 4), dst2))
        h = h + s_hg @ pi["W2"] + sg * pi["b2"]
        ci = params[conv_key]
        agg = _unslice(_gat_segsum(_slices(h, 4), src2, dst2))
        z = (agg * inv) @ ci["Wl"] + ci["bl"] + h @ ci["Wr"]
        h = jax.nn.relu(_bn(z, params[bn_key]))

    pi = params["inj3"]
    he = jax.nn.relu(edge_attr @ pi["W1"] + pi["b1"]) * g
    s_hg = _unslice(_lin_segsum(_slices(he, 4), dst2))
    h = h + s_hg @ pi["W2"] + sg * pi["b2"]
    ci = params["conv3"]
    y = h @ ci["Wl"]
    aggy = _unslice(_gat_segsum(_slices(y, 2), src2, dst2))
    z = aggy * inv + ci["bl"] + h @ ci["Wr"]
    h = jax.nn.relu(_bn(z, params["bn3"]))

    return _head(h, params)


# speedup vs baseline: 4.5886x; 2.7903x over previous
import functools

import jax
import jax.numpy as jnp
from jax import lax
from jax.experimental import pallas as pl
from jax.experimental.pallas import tpu as pltpu
from jax.experimental.pallas import tpu_sc as plsc

_N = 100000
_E = 1600000
_EP = 1638400
_R = _EP // 128
_HALF = _R // 2
_TROWS = _HALF // 16
_BATCH = 8
_NB = _TROWS // _BATCH
_NACC = 100096
_TACC = _NACC // 16
_ZCH = _TACC // 2
_RW = _BATCH * 128


def _sc_body(S, mode, refs):
    gather = mode == "gat"
    extras = mode == "mlp" and S == 1
    i = 0
    if gather:
        pays = refs[i:i + S]; i += S
        src2 = refs[i]; i += 1
    else:
        e0h, e1h, e2h, gh = refs[i:i + 4]; i += 4
        if extras:
            gth = refs[i]; i += 1
        wh = refs[i]; i += 1
        bh = refs[i]; i += 1
    dst2 = refs[i]; i += 1
    out = refs[i]; i += 1
    acc = refs[i]; i += 1
    dbuf = refs[i]; i += 1
    if gather:
        sbuf = refs[i]; i += 1
    else:
        e0b, e1b, e2b, gb = refs[i:i + 4]; i += 4
        if extras:
            gtb = refs[i]; i += 1
        wbuf = refs[i]; i += 1
        bbuf = refs[i]; i += 1
    rows = refs[i]; i += 1
    sem_g = refs[i]; i += 1
    sem_s = refs[i]; i += 1

    c = lax.axis_index("c")
    s = lax.axis_index("s")

    z16 = jnp.zeros((16,), jnp.float32)
    base = c * _HALF + s * _TROWS

    if not gather:
        pltpu.sync_copy(wh, wbuf)
        pltpu.sync_copy(bh, bbuf)
        if extras:
            lanes = lax.iota(jnp.int32, 16)
            oh5 = jnp.where(lanes == 5, 1.0, 0.0).astype(jnp.float32)
            oh6 = jnp.where(lanes == 6, 1.0, 0.0).astype(jnp.float32)

    for f in range(S):
        if gather:
            pay = pays[f]
        else:
            w0 = wbuf[3 * f + 0, :]
            w1 = wbuf[3 * f + 1, :]
            w2 = wbuf[3 * f + 2, :]
            bv = bbuf[f, :]

        def zb(k, carry):
            rows[k, :] = z16
            return carry

        lax.fori_loop(0, _RW, zb, 0)
        for k in range(6):
            pltpu.sync_copy(rows, acc.at[pl.ds(s * _TACC + k * _RW, _RW)])
        pltpu.sync_copy(rows.at[pl.ds(0, _TACC - 6 * _RW)],
                        acc.at[pl.ds(s * _TACC + 6 * _RW, _TACC - 6 * _RW)])
        plsc.subcore_barrier()

        def batch_body(b, carry):
            row0 = base + b * _BATCH
            e0 = row0 * 128
            pltpu.sync_copy(dst2.at[pl.ds(row0, _BATCH)], dbuf)
            if gather:
                pltpu.sync_copy(src2.at[pl.ds(row0, _BATCH)], sbuf)
                descs = [pltpu.async_copy(pay.at[sbuf.at[j]],
                                          rows.at[pl.ds(j * 128, 128)], sem_g)
                         for j in range(_BATCH)]
                for d in descs:
                    d.wait()
            else:
                pltpu.sync_copy(e0h.at[pl.ds(e0, _RW)], e0b)
                pltpu.sync_copy(e1h.at[pl.ds(e0, _RW)], e1b)
                pltpu.sync_copy(e2h.at[pl.ds(e0, _RW)], e2b)
                pltpu.sync_copy(gh.at[pl.ds(e0, _RW)], gb)
                if extras:
                    pltpu.sync_copy(gth.at[pl.ds(e0, _RW)], gtb)

                def mlp_body(k, carry):
                    ev0 = e0b[pl.ds(k * 16, 16)]
                    ev1 = e1b[pl.ds(k * 16, 16)]
                    ev2 = e2b[pl.ds(k * 16, 16)]
                    evg = gb[pl.ds(k * 16, 16)]
                    if extras:
                        evt = gtb[pl.ds(k * 16, 16)]
                    for u in range(16):
                        m = jnp.maximum(ev0[u] * w0 + ev1[u] * w1
                                        + ev2[u] * w2 + bv, 0.0) * evg[u]
                        if extras:
                            m = m + evt[u] * oh5 + evg[u] * oh6
                        rows[k * 16 + u, :] = m
                    return carry

                lax.fori_loop(0, _RW // 16, mlp_body, 0)
            descs = [pltpu.async_copy(rows.at[pl.ds(j * 128, 128)],
                                      acc.at[dbuf.at[j]], sem_s, add=True)
                     for j in range(_BATCH)]
            for d in descs:
                d.wait()
            return carry

        lax.fori_loop(0, _NB, batch_body, 0)
        plsc.subcore_barrier()

        off = (c * S + f) * _NACC + s * _TACC
        for k in range(2):
            pltpu.sync_copy(acc.at[pl.ds(s * _TACC + k * _ZCH, _ZCH)],
                            out.at[pl.ds(off + k * _ZCH, _ZCH)])
        plsc.subcore_barrier()


_SC_CACHE = {}


def _sc_segsum(S, mode):
    key = (S, mode)
    if key not in _SC_CACHE:
        gather = mode == "gat"
        extras = mode == "mlp" and S == 1
        scratch = [pltpu.VMEM_SHARED((_NACC, 16), jnp.float32),
                   pltpu.VMEM((_BATCH, 128), jnp.int32)]
        if gather:
            scratch.append(pltpu.VMEM((_BATCH, 128), jnp.int32))
        else:
            scratch += [pltpu.VMEM((_RW,), jnp.float32)] * (5 if extras else 4)
            scratch += [pltpu.VMEM((3 * S, 16), jnp.float32),
                        pltpu.VMEM((S, 16), jnp.float32)]
        scratch += [pltpu.VMEM((_BATCH * 128, 16), jnp.float32),
                    pltpu.SemaphoreType.DMA,
                    pltpu.SemaphoreType.DMA]
        fn = pl.kernel(
            lambda *refs: _sc_body(S, mode, refs),
            out_type=jax.ShapeDtypeStruct((2 * S * _NACC, 16), jnp.float32),
            mesh=plsc.VectorSubcoreMesh(core_axis_name="c",
                                        subcore_axis_name="s"),
            scratch_types=scratch,
            compiler_params=pltpu.CompilerParams(use_tc_tiling_on_sc=False),
            name=f"segsum_s{S}_{mode}",
        )
        _SC_CACHE[key] = fn
    return _SC_CACHE[key]


def _merge(part, S):
    return part.reshape(2, S, _NACC, 16)[:, :, :_N].sum(axis=0)


def _mlp_segsum(S, ecols, g, w1, b1, dst2, gate=None):
    w = w1.reshape(3, S, 16).transpose(1, 0, 2).reshape(3 * S, 16)
    b = b1.reshape(S, 16)
    args = list(ecols) + [g]
    if gate is not None:
        args.append(gate)
    args += [w, b, dst2]
    return _merge(_sc_segsum(S, "mlp")(*args), S)


def _gat_segsum(tables, src2, dst2):
    S = len(tables)
    part = _sc_segsum(S, "gat")(*tables, src2, dst2)
    return _merge(part, S)


def _slices(m, S):
    return [m[:, 16 * f:16 * f + 16] for f in range(S)]


def _unslice(t):
    return t.transpose(1, 0, 2).reshape(t.shape[1], -1)


def _bn(z, p, eps=1e-5):
    m = jnp.mean(z, axis=0)
    v = jnp.var(z, axis=0)
    return p["g"] * (z - m) / jnp.sqrt(v + eps) + p["b"]


def _head_kernel(h_ref, ws_ref, bs_ref, wf1_ref, bf1_ref, wf2_ref, bf2_ref,
                 wr1_ref, br1_ref, wr2_ref, br2_ref, fire_ref, rate_ref):
    h = h_ref[...]
    sh = jnp.maximum(h @ ws_ref[...] + bs_ref[...], 0.0)
    f = jnp.maximum(sh @ wf1_ref[...] + bf1_ref[...], 0.0)
    fire = f @ wf2_ref[...] + bf2_ref[...]
    r = jnp.maximum(sh @ wr1_ref[...] + br1_ref[...], 0.0)
    rate = jnp.maximum(r @ wr2_ref[...] + br2_ref[...], 0.0)
    fire_ref[...] = fire
    rate_ref[...] = rate


def _head(h, params):
    n = h.shape[0]
    blk = 10000
    grid = (n // blk,)
    out_shape = (jax.ShapeDtypeStruct((n, 1), jnp.float32),
                 jax.ShapeDtypeStruct((n, 1), jnp.float32))
    wmap = lambda i: (0, 0)
    bmap = lambda i: (0,)
    fire, rate = pl.pallas_call(
        _head_kernel,
        grid=grid,
        in_specs=[pl.BlockSpec((blk, h.shape[1]), lambda i: (i, 0)),
                  pl.BlockSpec((32, 16), wmap),
                  pl.BlockSpec((16,), bmap),
                  pl.BlockSpec((16, 8), wmap),
                  pl.BlockSpec((8,), bmap),
                  pl.BlockSpec((8, 1), wmap),
                  pl.BlockSpec((1,), bmap),
                  pl.BlockSpec((16, 8), wmap),
                  pl.BlockSpec((8,), bmap),
                  pl.BlockSpec((8, 1), wmap),
                  pl.BlockSpec((1,), bmap)],
        out_specs=(pl.BlockSpec((blk, 1), lambda i: (i, 0)),
                   pl.BlockSpec((blk, 1), lambda i: (i, 0))),
        out_shape=out_shape,
    )(h, params["shared"]["W"], params["shared"]["b"],
      params["fire1"]["W"], params["fire1"]["b"],
      params["fire2"]["W"], params["fire2"]["b"],
      params["rate1"]["W"], params["rate1"]["b"],
      params["rate2"]["W"], params["rate2"]["b"])
    return fire[:, 0], rate[:, 0]


def kernel(x, edge_index, edge_attr, params):
    src, dst = edge_index[0], edge_index[1]
    npad = _EP - _E
    dst_pad = _N + jnp.arange(npad, dtype=jnp.int32) % (_NACC - _N)
    src_pad = jnp.arange(npad, dtype=jnp.int32) % _N
    src2 = jnp.concatenate([src, src_pad]).reshape(_R, 128)
    dst2 = jnp.concatenate([dst, dst_pad]).reshape(_R, 128)

    zpad = jnp.zeros((npad,), jnp.float32)
    ecols = [jnp.concatenate([edge_attr[:, i], zpad]) for i in range(3)]
    g1 = jax.nn.sigmoid(edge_attr[:, 2])
    gp = jnp.concatenate([g1, zpad])
    gate = jnp.concatenate([jnp.ones((_E,), jnp.float32), zpad])

    p1 = params["inj1"]
    w1p = jnp.zeros((3, 16), jnp.float32).at[:, :5].set(p1["W1"])
    b1p = jnp.zeros((16,), jnp.float32).at[:5].set(p1["b1"])
    acc1 = _mlp_segsum(1, ecols, gp, w1p, b1p, dst2, gate=gate)[0]
    s_hg1 = acc1[:, :5]
    cnt = acc1[:, 5]
    sg = acc1[:, 6:7]
    inv = (1.0 / jnp.maximum(cnt, 1.0))[:, None]

    h1 = x + s_hg1 @ p1["W2"] + sg * p1["b2"]
    c1 = params["conv1"]
    tab1 = jnp.pad(h1, ((0, 0), (0, 11)))
    agg1 = _gat_segsum([tab1], src2, dst2)[0][:, :5]
    z = (agg1 * inv) @ c1["Wl"] + c1["bl"] + h1 @ c1["Wr"]
    h = jax.nn.relu(_bn(z, params["bn1"]))

    for inj_key, conv_key, bn_key in (("inj2", "conv2", "bn2"),):
        pi = params[inj_key]
        s_hg = _unslice(_mlp_segsum(4, ecols, gp, pi["W1"], pi["b1"], dst2))
        h = h + s_hg @ pi["W2"] + sg * pi["b2"]
        ci = params[conv_key]
        agg = _unslice(_gat_segsum(_slices(h, 4), src2, dst2))
        z = (agg * inv) @ ci["Wl"] + ci["bl"] + h @ ci["Wr"]
        h = jax.nn.relu(_bn(z, params[bn_key]))

    pi = params["inj3"]
    s_hg = _unslice(_mlp_segsum(4, ecols, gp, pi["W1"], pi["b1"], dst2))
    h = h + s_hg @ pi["W2"] + sg * pi["b2"]
    ci = params["conv3"]
    y = h @ ci["Wl"]
    aggy = _unslice(_gat_segsum(_slices(y, 2), src2, dst2))
    z = aggy * inv + ci["bl"] + h @ ci["Wr"]
    h = jax.nn.relu(_bn(z, params["bn3"]))

    return _head(h, params)


# speedup vs baseline: 4.8674x; 1.0608x over previous
import functools

import jax
import jax.numpy as jnp
from jax import lax
from jax.experimental import pallas as pl
from jax.experimental.pallas import tpu as pltpu
from jax.experimental.pallas import tpu_sc as plsc

_N = 100000
_E = 1600000
_EP = 1638400
_R = _EP // 128
_HALF = _R // 2
_TROWS = _HALF // 16
_BATCH = 8
_NB = _TROWS // _BATCH
_NACC = 100096
_TACC = _NACC // 16
_ZCH = _TACC // 2
_RW = _BATCH * 128


def _sc_body(S, mode, extras, refs):
    gather = mode == "gat"
    i = 0
    if gather:
        pays = refs[i:i + S]; i += S
        src2 = refs[i]; i += 1
    else:
        e0h, e1h, e2h, gh = refs[i:i + 4]; i += 4
        if extras:
            gth = refs[i]; i += 1
        wh = refs[i]; i += 1
        bh = refs[i]; i += 1
    dst2 = refs[i]; i += 1
    out = refs[i]; i += 1
    acc = refs[i]; i += 1
    dbuf = refs[i]; i += 1
    if gather:
        sbuf = refs[i]; i += 1
    else:
        e0b, e1b, e2b, gb = refs[i:i + 4]; i += 4
        if extras:
            gtb = refs[i]; i += 1
        wbuf = refs[i]; i += 1
        bbuf = refs[i]; i += 1
    rows = refs[i]; i += 1
    sem_g = refs[i]; i += 1
    sem_s = refs[i]; i += 1

    c = lax.axis_index("c")
    s = lax.axis_index("s")

    z16 = jnp.zeros((16,), jnp.float32)
    base = c * _HALF + s * _TROWS

    if not gather:
        pltpu.sync_copy(wh, wbuf)
        pltpu.sync_copy(bh, bbuf)
        if extras:
            lanes = lax.iota(jnp.int32, 16)
            oh5 = jnp.where(lanes == 5, 1.0, 0.0).astype(jnp.float32)
            oh6 = jnp.where(lanes == 6, 1.0, 0.0).astype(jnp.float32)

    for f in range(S):
        if gather:
            pay = pays[f]
        else:
            w0 = wbuf[3 * f + 0, :]
            w1 = wbuf[3 * f + 1, :]
            w2 = wbuf[3 * f + 2, :]
            bv = bbuf[f, :]

        def zb(k, carry):
            rows[k, :] = z16
            return carry

        lax.fori_loop(0, _RW, zb, 0)
        for k in range(6):
            pltpu.sync_copy(rows, acc.at[pl.ds(s * _TACC + k * _RW, _RW)])
        pltpu.sync_copy(rows.at[pl.ds(0, _TACC - 6 * _RW)],
                        acc.at[pl.ds(s * _TACC + 6 * _RW, _TACC - 6 * _RW)])
        plsc.subcore_barrier()

        def batch_body(b, carry):
            row0 = base + b * _BATCH
            e0 = row0 * 128
            pltpu.sync_copy(dst2.at[pl.ds(row0, _BATCH)], dbuf)
            if gather:
                pltpu.sync_copy(src2.at[pl.ds(row0, _BATCH)], sbuf)
                descs = [pltpu.async_copy(pay.at[sbuf.at[j]],
                                          rows.at[pl.ds(j * 128, 128)], sem_g)
                         for j in range(_BATCH)]
                for d in descs:
                    d.wait()
            else:
                pltpu.sync_copy(e0h.at[pl.ds(e0, _RW)], e0b)
                pltpu.sync_copy(e1h.at[pl.ds(e0, _RW)], e1b)
                pltpu.sync_copy(e2h.at[pl.ds(e0, _RW)], e2b)
                pltpu.sync_copy(gh.at[pl.ds(e0, _RW)], gb)
                if extras:
                    pltpu.sync_copy(gth.at[pl.ds(e0, _RW)], gtb)

                def mlp_body(k, carry):
                    ev0 = e0b[pl.ds(k * 16, 16)]
                    ev1 = e1b[pl.ds(k * 16, 16)]
                    ev2 = e2b[pl.ds(k * 16, 16)]
                    evg = gb[pl.ds(k * 16, 16)]
                    if extras:
                        evt = gtb[pl.ds(k * 16, 16)]
                    for u in range(16):
                        m = jnp.maximum(ev0[u] * w0 + ev1[u] * w1
                                        + ev2[u] * w2 + bv, 0.0) * evg[u]
                        if extras:
                            m = m + evt[u] * oh5 + evg[u] * oh6
                        rows[k * 16 + u, :] = m
                    return carry

                lax.fori_loop(0, _RW // 16, mlp_body, 0)
            descs = [pltpu.async_copy(rows.at[pl.ds(j * 128, 128)],
                                      acc.at[dbuf.at[j]], sem_s, add=True)
                     for j in range(_BATCH)]
            for d in descs:
                d.wait()
            return carry

        lax.fori_loop(0, _NB, batch_body, 0)
        plsc.subcore_barrier()

        off = (c * S + f) * _NACC + s * _TACC
        for k in range(2):
            pltpu.sync_copy(acc.at[pl.ds(s * _TACC + k * _ZCH, _ZCH)],
                            out.at[pl.ds(off + k * _ZCH, _ZCH)])
        plsc.subcore_barrier()


_SC_CACHE = {}


def _sc_segsum(S, mode, extras=False):
    key = (S, mode, extras)
    if key not in _SC_CACHE:
        gather = mode == "gat"
        scratch = [pltpu.VMEM_SHARED((_NACC, 16), jnp.float32),
                   pltpu.VMEM((_BATCH, 128), jnp.int32)]
        if gather:
            scratch.append(pltpu.VMEM((_BATCH, 128), jnp.int32))
        else:
            scratch += [pltpu.VMEM((_RW,), jnp.float32)] * (5 if extras else 4)
            scratch += [pltpu.VMEM((3 * S, 16), jnp.float32),
                        pltpu.VMEM((S, 16), jnp.float32)]
        scratch += [pltpu.VMEM((_BATCH * 128, 16), jnp.float32),
                    pltpu.SemaphoreType.DMA,
                    pltpu.SemaphoreType.DMA]
        fn = pl.kernel(
            lambda *refs: _sc_body(S, mode, extras, refs),
            out_type=jax.ShapeDtypeStruct((2 * S * _NACC, 16), jnp.float32),
            mesh=plsc.VectorSubcoreMesh(core_axis_name="c",
                                        subcore_axis_name="s"),
            scratch_types=scratch,
            compiler_params=pltpu.CompilerParams(use_tc_tiling_on_sc=False),
            name=f"segsum_s{S}_{mode}",
        )
        _SC_CACHE[key] = fn
    return _SC_CACHE[key]


def _merge(part, S):
    return part.reshape(2, S, _NACC, 16)[:, :, :_N].sum(axis=0)


def _after(x, dep):
    x, _ = lax.optimization_barrier((x, dep))
    return x


def _mlp_segsum(S, ecols, g, w1, b1, dst2, gate=None):
    w = w1.reshape(3, S, 16).transpose(1, 0, 2).reshape(3 * S, 16)
    b = b1.reshape(S, 16)
    args = list(ecols) + [g]
    if gate is not None:
        args.append(gate)
    args += [w, b, dst2]
    part = _sc_segsum(S, "mlp", gate is not None)(*args)
    return _merge(part, S), part


def _gat_segsum(tables, src2, dst2):
    S = len(tables)
    part = _sc_segsum(S, "gat")(*tables, src2, dst2)
    return _merge(part, S), part


def _slices(m, S):
    return [m[:, 16 * f:16 * f + 16] for f in range(S)]


def _unslice(t):
    return t.transpose(1, 0, 2).reshape(t.shape[1], -1)


def _bn(z, p, eps=1e-5):
    m = jnp.mean(z, axis=0)
    v = jnp.var(z, axis=0)
    return p["g"] * (z - m) / jnp.sqrt(v + eps) + p["b"]


def _head_kernel(h_ref, ws_ref, bs_ref, wf1_ref, bf1_ref, wf2_ref, bf2_ref,
                 wr1_ref, br1_ref, wr2_ref, br2_ref, fire_ref, rate_ref):
    h = h_ref[...]
    sh = jnp.maximum(h @ ws_ref[...] + bs_ref[...], 0.0)
    f = jnp.maximum(sh @ wf1_ref[...] + bf1_ref[...], 0.0)
    fire = f @ wf2_ref[...] + bf2_ref[...]
    r = jnp.maximum(sh @ wr1_ref[...] + br1_ref[...], 0.0)
    rate = jnp.maximum(r @ wr2_ref[...] + br2_ref[...], 0.0)
    fire_ref[...] = fire
    rate_ref[...] = rate


def _head(h, params):
    n = h.shape[0]
    blk = 10000
    grid = (n // blk,)
    out_shape = (jax.ShapeDtypeStruct((n, 1), jnp.float32),
                 jax.ShapeDtypeStruct((n, 1), jnp.float32))
    wmap = lambda i: (0, 0)
    bmap = lambda i: (0,)
    fire, rate = pl.pallas_call(
        _head_kernel,
        grid=grid,
        in_specs=[pl.BlockSpec((blk, h.shape[1]), lambda i: (i, 0)),
                  pl.BlockSpec((32, 16), wmap),
                  pl.BlockSpec((16,), bmap),
                  pl.BlockSpec((16, 8), wmap),
                  pl.BlockSpec((8,), bmap),
                  pl.BlockSpec((8, 1), wmap),
                  pl.BlockSpec((1,), bmap),
                  pl.BlockSpec((16, 8), wmap),
                  pl.BlockSpec((8,), bmap),
                  pl.BlockSpec((8, 1), wmap),
                  pl.BlockSpec((1,), bmap)],
        out_specs=(pl.BlockSpec((blk, 1), lambda i: (i, 0)),
                   pl.BlockSpec((blk, 1), lambda i: (i, 0))),
        out_shape=out_shape,
    )(h, params["shared"]["W"], params["shared"]["b"],
      params["fire1"]["W"], params["fire1"]["b"],
      params["fire2"]["W"], params["fire2"]["b"],
      params["rate1"]["W"], params["rate1"]["b"],
      params["rate2"]["W"], params["rate2"]["b"])
    return fire[:, 0], rate[:, 0]


def kernel(x, edge_index, edge_attr, params):
    src, dst = edge_index[0], edge_index[1]
    npad = _EP - _E
    dst_pad = _N + jnp.arange(npad, dtype=jnp.int32) % (_NACC - _N)
    src_pad = jnp.arange(npad, dtype=jnp.int32) % _N
    src2 = jnp.concatenate([src, src_pad]).reshape(_R, 128)
    dst2 = jnp.concatenate([dst, dst_pad]).reshape(_R, 128)

    zpad = jnp.zeros((npad,), jnp.float32)
    ecols = [jnp.concatenate([edge_attr[:, i], zpad]) for i in range(3)]
    g1 = jax.nn.sigmoid(edge_attr[:, 2])
    gp = jnp.concatenate([g1, zpad])
    gate = jnp.concatenate([jnp.ones((_E,), jnp.float32), zpad])

    p1 = params["inj1"]
    w1p = jnp.zeros((3, 16), jnp.float32).at[:, :5].set(p1["W1"])
    b1p = jnp.zeros((16,), jnp.float32).at[:5].set(p1["b1"])
    acc1, r1 = _mlp_segsum(1, ecols, gp, w1p, b1p, dst2, gate=gate)
    acc1 = acc1[0]
    s_hg1 = acc1[:, :5]
    cnt = acc1[:, 5]
    sg = acc1[:, 6:7]
    inv = (1.0 / jnp.maximum(cnt, 1.0))[:, None]

    pi2 = params["inj2"]
    a2a, r2a = _mlp_segsum(1, ecols, gp, _after(pi2["W1"][:, :16], r1),
                           pi2["b1"][:16], dst2)
    a2a = a2a[0]

    h1 = x + s_hg1 @ p1["W2"] + sg * p1["b2"]
    c1 = params["conv1"]
    tab1 = _after(jnp.pad(h1, ((0, 0), (0, 11))), r2a)
    agg1, rg1 = _gat_segsum([tab1], src2, dst2)
    agg1 = agg1[0][:, :5]

    a2b, r2b = _mlp_segsum(3, ecols, gp, _after(pi2["W1"][:, 16:], rg1),
                           pi2["b1"][16:], dst2)

    z = (agg1 * inv) @ c1["Wl"] + c1["bl"] + h1 @ c1["Wr"]
    h = jax.nn.relu(_bn(z, params["bn1"]))

    s_hg = jnp.concatenate([a2a, _unslice(a2b)], axis=1)
    h = h + s_hg @ pi2["W2"] + sg * pi2["b2"]

    pi3 = params["inj3"]
    a3a, r3a = _mlp_segsum(2, ecols, gp, _after(pi3["W1"][:, :32], r2b),
                           pi3["b1"][:32], dst2)

    ci2 = params["conv2"]
    tabs = _slices(h, 4)
    tabs[0] = _after(tabs[0], r3a)
    agg, rg2 = _gat_segsum(tabs, src2, dst2)
    agg = _unslice(agg)

    a3b, r3b = _mlp_segsum(2, ecols, gp, _after(pi3["W1"][:, 32:], rg2),
                           pi3["b1"][32:], dst2)

    z = (agg * inv) @ ci2["Wl"] + ci2["bl"] + h @ ci2["Wr"]
    h = jax.nn.relu(_bn(z, params["bn2"]))

    s_hg = jnp.concatenate([_unslice(a3a), _unslice(a3b)], axis=1)
    h = h + s_hg @ pi3["W2"] + sg * pi3["b2"]
    ci = params["conv3"]
    y = h @ ci["Wl"]
    aggy = _unslice(_gat_segsum(_slices(y, 2), src2, dst2)[0])
    z = aggy * inv + ci["bl"] + h @ ci["Wr"]
    h = jax.nn.relu(_bn(z, params["bn3"]))

    return _head(h, params)
